# Initial kernel scaffold; baseline (speedup 1.0000x reference)
import jax
import jax.numpy as jnp
from jax.experimental import pallas as pl


def kernel(users, items, user_table, item_table, W, b):
    raise NotImplementedError("write your pallas kernel here")


# speedup vs baseline: 1.0941x; 1.0941x over previous
import functools

import jax
import jax.numpy as jnp
from jax import lax
from jax.experimental import pallas as pl
from jax.experimental.pallas import tpu as pltpu
from jax.experimental.pallas import tpu_sc as plsc

D = 32
L = 200
LP = 208
NCH = 2
CH = LP // NCH
NG = LP // 16
NC = 2
NS = 16
NW = NC * NS


def _body(users_hbm, items_hbm, utab_hbm, itab_hbm, wb_hbm, out_hbm,
          uidx_v, urows_v, vw_v, wb_v, idx_v, rows_v, out_v, sem):
    wid = lax.axis_index("s") * NC + lax.axis_index("c")
    upw = users_hbm.shape[0] // NW
    base_u = wid * upw

    pltpu.sync_copy(users_hbm.at[pl.ds(base_u, upw)], uidx_v)
    pltpu.sync_copy(wb_hbm, wb_v)
    pltpu.async_copy(utab_hbm.at[uidx_v], urows_v, sem).wait()
    w0 = wb_v[pl.ds(0, 16)]
    w1 = wb_v[pl.ds(16, 16)]
    bias_vec = jnp.broadcast_to(wb_v[pl.ds(32, 16)][0], (16,))

    def init_vw(u, _):
        vw_v[u, pl.ds(0, 16)] = urows_v[u, pl.ds(0, 16)] * w0
        vw_v[u, pl.ds(16, 16)] = urows_v[u, pl.ds(16, 16)] * w1
        return _
    lax.fori_loop(0, upw, init_vw, None)

    lanes = lax.iota(jnp.int32, 16)

    def per_user(u, _):
        gu = base_u + u
        pltpu.sync_copy(items_hbm.at[gu], idx_v)
        cp0 = pltpu.async_copy(itab_hbm.at[idx_v.at[0]], rows_v.at[pl.ds(0, CH)], sem)
        cp1 = pltpu.async_copy(itab_hbm.at[idx_v.at[1]], rows_v.at[pl.ds(CH, CH)], sem)
        cp0.wait()
        cp1.wait()

        accs = [bias_vec] * NG
        row_of = [lanes + (16 * g) for g in range(NG)]
        vwa = vw_v[u, pl.ds(0, 16)]
        vwb = vw_v[u, pl.ds(16, 16)]
        for d in range(D):
            sd = vwa[d] if d < 16 else vwb[d - 16]
            dvec = jnp.broadcast_to(jnp.int32(d), (16,))
            for g in range(NG):
                col = plsc.load_gather(rows_v, [row_of[g], dvec])
                accs[g] = accs[g] + col * sd
        for g in range(NG):
            out_v[pl.ds(16 * g, 16)] = accs[g]
        pltpu.sync_copy(out_v, out_hbm.at[gu])
        return _

    lax.fori_loop(0, upw, per_user, None)


@jax.jit
def kernel(users, items, user_table, item_table, W, b):
    B = users.shape[0]
    users = users.astype(jnp.int32)
    items = items.astype(jnp.int32)
    items_p = jnp.concatenate(
        [items, jnp.zeros((B, LP - L), jnp.int32)], axis=1).reshape(B, NCH, CH)
    wb = jnp.concatenate(
        [W.reshape(-1), b.reshape(-1), jnp.zeros((15,), jnp.float32)])

    run = pl.kernel(
        _body,
        out_type=jax.ShapeDtypeStruct((B, LP), jnp.float32),
        mesh=plsc.VectorSubcoreMesh(
            core_axis_name="c", subcore_axis_name="s",
            num_cores=NC, num_subcores=NS),
        compiler_params=pltpu.CompilerParams(
            use_tc_tiling_on_sc=False, needs_layout_passes=False),
        scratch_types=[
            pltpu.VMEM((B // NW,), jnp.int32),
            pltpu.VMEM((B // NW, D), jnp.float32),
            pltpu.VMEM((B // NW, D), jnp.float32),
            pltpu.VMEM((48,), jnp.float32),
            pltpu.VMEM((NCH, CH), jnp.int32),
            pltpu.VMEM((LP, D), jnp.float32),
            pltpu.VMEM((LP,), jnp.float32),
            pltpu.SemaphoreType.DMA,
        ],
    )
    out = run(users, items_p, user_table, item_table, wb)
    return out[:, :L]


# speedup vs baseline: 1.1875x; 1.0854x over previous
import functools

import jax
import jax.numpy as jnp
from jax import lax
from jax.experimental import pallas as pl
from jax.experimental.pallas import tpu as pltpu
from jax.experimental.pallas import tpu_sc as plsc

D = 32
L = 200
LP = 208
NCH = 2
CH = LP // NCH
NG = LP // 16
NC = 2
NS = 16
NW = NC * NS
NBUF = 4


def _body(users_hbm, items_hbm, utab_hbm, itab_hbm, wb_hbm, out_hbm,
          uidx_v, urows_v, vw_v, wb_v, idx_v, rows_v, out_v,
          sem, gsem0, gsem1, gsem2, gsem3):
    gsems = [gsem0, gsem1, gsem2, gsem3]
    wid = lax.axis_index("s") * NC + lax.axis_index("c")
    upw = users_hbm.shape[0] // NW
    base_u = wid * upw

    pltpu.sync_copy(users_hbm.at[pl.ds(base_u, upw)], uidx_v)
    pltpu.sync_copy(items_hbm.at[pl.ds(base_u, upw)], idx_v)
    pltpu.sync_copy(wb_hbm, wb_v)
    pltpu.async_copy(utab_hbm.at[uidx_v], urows_v, sem).wait()
    w0 = wb_v[pl.ds(0, 16)]
    w1 = wb_v[pl.ds(16, 16)]
    bias_vec = jnp.broadcast_to(wb_v[pl.ds(32, 16)][0], (16,))

    def init_vw(u, _):
        vw_v[u, pl.ds(0, 16)] = urows_v[u, pl.ds(0, 16)] * w0
        vw_v[u, pl.ds(16, 16)] = urows_v[u, pl.ds(16, 16)] * w1
        return _
    lax.fori_loop(0, upw, init_vw, None)

    lanes = lax.iota(jnp.int32, 16)
    row_of = [lanes + (16 * g) for g in range(NG)]

    def issue(u, b):
        pltpu.async_copy(itab_hbm.at[idx_v.at[u, 0]],
                         rows_v.at[b, pl.ds(0, CH)], gsems[b])
        pltpu.async_copy(itab_hbm.at[idx_v.at[u, 1]],
                         rows_v.at[b, pl.ds(CH, CH)], gsems[b])

    def wait_rows(b):
        pltpu.make_async_copy(itab_hbm.at[pl.ds(0, LP)],
                              rows_v.at[b], gsems[b]).wait()

    def compute(u, b):
        accs = [bias_vec] * NG
        vwa = vw_v[u, pl.ds(0, 16)]
        vwb = vw_v[u, pl.ds(16, 16)]
        for d in range(D):
            sd = vwa[d] if d < 16 else vwb[d - 16]
            dvec = jnp.broadcast_to(jnp.int32(d), (16,))
            for g in range(NG):
                col = plsc.load_gather(rows_v.at[b], [row_of[g], dvec])
                accs[g] = accs[g] + col * sd
        for g in range(NG):
            out_v[u, pl.ds(16 * g, 16)] = accs[g]

    for b in range(NBUF):
        issue(b, b)

    def step(i, _):
        for b in range(NBUF):
            u = i * NBUF + b
            wait_rows(b)
            compute(u, b)
            nu = u + NBUF

            @pl.when(nu < upw)
            def _():
                issue(nu, b)
        return _

    lax.fori_loop(0, upw // NBUF, step, None)
    pltpu.sync_copy(out_v, out_hbm.at[pl.ds(base_u, upw)])


@jax.jit
def kernel(users, items, user_table, item_table, W, b):
    B = users.shape[0]
    users = users.astype(jnp.int32)
    items = items.astype(jnp.int32)
    items_p = jnp.concatenate(
        [items, jnp.zeros((B, LP - L), jnp.int32)], axis=1).reshape(B, NCH, CH)
    wb = jnp.concatenate(
        [W.reshape(-1), b.reshape(-1), jnp.zeros((15,), jnp.float32)])

    run = pl.kernel(
        _body,
        out_type=jax.ShapeDtypeStruct((B, LP), jnp.float32),
        mesh=plsc.VectorSubcoreMesh(
            core_axis_name="c", subcore_axis_name="s",
            num_cores=NC, num_subcores=NS),
        compiler_params=pltpu.CompilerParams(
            use_tc_tiling_on_sc=False, needs_layout_passes=False),
        scratch_types=[
            pltpu.VMEM((B // NW,), jnp.int32),
            pltpu.VMEM((B // NW, D), jnp.float32),
            pltpu.VMEM((B // NW, D), jnp.float32),
            pltpu.VMEM((48,), jnp.float32),
            pltpu.VMEM((B // NW, NCH, CH), jnp.int32),
            pltpu.VMEM((NBUF, LP, D), jnp.float32),
            pltpu.VMEM((B // NW, LP), jnp.float32),
            pltpu.SemaphoreType.DMA,
            pltpu.SemaphoreType.DMA,
            pltpu.SemaphoreType.DMA,
            pltpu.SemaphoreType.DMA,
            pltpu.SemaphoreType.DMA,
        ],
    )
    out = run(users, items_p, user_table, item_table, wb)
    return out[:, :L]


# speedup vs baseline: 1.5694x; 1.3216x over previous
import functools

import jax
import jax.numpy as jnp
from jax import lax
from jax.experimental import pallas as pl
from jax.experimental.pallas import tpu as pltpu
from jax.experimental.pallas import tpu_sc as plsc

D = 32
L = 200
LP = 208
NCH = 4
CH = LP // NCH
NG = LP // 16
NC = 2
NS = 16
NW = NC * NS
NBUF = 6


def _body(items_hbm, itab_hbm, vw_hbm, bias_hbm, out_hbm,
          vw_v, bias_v, idx_v, rows_v, out_v, *gsems):
    wid = lax.axis_index("s") * NC + lax.axis_index("c")
    upw = vw_hbm.shape[0] // NW
    base_u = wid * upw

    pltpu.sync_copy(items_hbm.at[pl.ds(base_u, upw)], idx_v)
    pltpu.sync_copy(vw_hbm.at[pl.ds(base_u, upw)], vw_v)
    pltpu.sync_copy(bias_hbm, bias_v)
    bias_vec = bias_v[...]

    lanes = lax.iota(jnp.int32, 16)

    def issue(u, b):
        for c in range(NCH):
            pltpu.async_copy(itab_hbm.at[idx_v.at[u, c]],
                             rows_v.at[b, pl.ds(c * CH, CH)], gsems[b])

    def wait_rows(b):
        pltpu.make_async_copy(itab_hbm.at[pl.ds(0, LP)],
                              rows_v.at[b], gsems[b]).wait()

    def compute(u, b):
        vwa = vw_v[u, pl.ds(0, 16)]
        vwb = vw_v[u, pl.ds(16, 16)]
        accs = [bias_vec] * NG
        for d in range(D):
            sd = vwa[d] if d < 16 else vwb[d - 16]
            dvec = jnp.broadcast_to(jnp.int32(d), (16,))
            for g in range(NG):
                col = plsc.load_gather(rows_v.at[b], [lanes + (16 * g), dvec])
                accs[g] = accs[g] + col * sd
        for g in range(NG):
            out_v[u, pl.ds(16 * g, 16)] = accs[g]

    for b in range(NBUF):
        issue(b, b)

    def step(i, _):
        for b in range(NBUF):
            u = i * NBUF + b
            wait_rows(b)
            compute(u, b)
            nu = u + NBUF

            @pl.when(nu < upw)
            def _():
                issue(nu, b)
        return _

    lax.fori_loop(0, upw // NBUF, step, None)
    for u in range((upw // NBUF) * NBUF, upw):
        b = u - (upw // NBUF) * NBUF
        wait_rows(b)
        compute(u, b)
    pltpu.sync_copy(out_v, out_hbm.at[pl.ds(base_u, upw)])


@jax.jit
def kernel(users, items, user_table, item_table, W, b):
    B = users.shape[0]
    items = items.astype(jnp.int32)
    vw = jnp.take(user_table, users, axis=0) * W.reshape(1, D)
    items_p = jnp.concatenate(
        [items, jnp.zeros((B, LP - L), jnp.int32)], axis=1).reshape(B, NCH, CH)
    bias16 = jnp.broadcast_to(b.reshape(()), (16,)).astype(jnp.float32)

    run = pl.kernel(
        _body,
        out_type=jax.ShapeDtypeStruct((B, LP), jnp.float32),
        mesh=plsc.VectorSubcoreMesh(
            core_axis_name="c", subcore_axis_name="s",
            num_cores=NC, num_subcores=NS),
        compiler_params=pltpu.CompilerParams(
            use_tc_tiling_on_sc=False, needs_layout_passes=False),
        scratch_types=[
            pltpu.VMEM((B // NW, D), jnp.float32),
            pltpu.VMEM((16,), jnp.float32),
            pltpu.VMEM((B // NW, NCH, CH), jnp.int32),
            pltpu.VMEM((NBUF, LP, D), jnp.float32),
            pltpu.VMEM((B // NW, LP), jnp.float32),
        ] + [pltpu.SemaphoreType.DMA] * NBUF,
    )
    out = run(items_p, item_table, vw, bias16)
    return out[:, :L]


# speedup vs baseline: 1.6010x; 1.0202x over previous
import functools

import jax
import jax.numpy as jnp
from jax import lax
from jax.experimental import pallas as pl
from jax.experimental.pallas import tpu as pltpu
from jax.experimental.pallas import tpu_sc as plsc

D = 32
L = 200
NCH = 5
CH = L // NCH
NG = 13
NC = 2
NS = 16
NW = NC * NS
NBUF = 6


def _body(items_hbm, itab_hbm, vw_hbm, bias_hbm, out_hbm,
          vw_v, bias_v, idx_v, rows_v, out_v, *gsems):
    wid = lax.axis_index("s") * NC + lax.axis_index("c")
    upw = vw_hbm.shape[0] // NW
    base_u = wid * upw

    pltpu.sync_copy(items_hbm.at[pl.ds(base_u * L, upw * L)], idx_v)
    pltpu.sync_copy(vw_hbm.at[pl.ds(base_u, upw)], vw_v)
    pltpu.sync_copy(bias_hbm, bias_v)
    bias_vec = bias_v[...]

    lanes = lax.iota(jnp.int32, 16)

    def issue(u, b):
        for c in range(NCH):
            pltpu.async_copy(itab_hbm.at[idx_v.at[pl.ds(u * L + c * CH, CH)]],
                             rows_v.at[b, pl.ds(c * CH, CH)], gsems[b])

    def wait_rows(b):
        pltpu.make_async_copy(itab_hbm.at[pl.ds(0, L)],
                              rows_v.at[b], gsems[b]).wait()

    def compute(u, b):
        vwa = vw_v[u, pl.ds(0, 16)]
        vwb = vw_v[u, pl.ds(16, 16)]
        offs = [16 * g for g in range(NG - 1)] + [L - 16]
        accs = [bias_vec] * NG
        for d in range(D):
            sd = vwa[d] if d < 16 else vwb[d - 16]
            dvec = jnp.broadcast_to(jnp.int32(d), (16,))
            for g in range(NG):
                col = plsc.load_gather(rows_v.at[b], [lanes + offs[g], dvec])
                accs[g] = accs[g] + col * sd
        for g in range(NG):
            out_v[u, pl.ds(offs[g], 16)] = accs[g]

    for b in range(NBUF):
        issue(b, b)

    def step(i, _):
        for b in range(NBUF):
            u = i * NBUF + b
            wait_rows(b)
            compute(u, b)
            nu = u + NBUF

            @pl.when(nu < upw)
            def _():
                issue(nu, b)
        return _

    lax.fori_loop(0, upw // NBUF, step, None)
    for u in range((upw // NBUF) * NBUF, upw):
        b = u - (upw // NBUF) * NBUF
        wait_rows(b)
        compute(u, b)
    pltpu.sync_copy(out_v, out_hbm.at[pl.ds(base_u, upw)])


@jax.jit
def kernel(users, items, user_table, item_table, W, b):
    B = users.shape[0]
    items_f = items.astype(jnp.int32).reshape(-1)
    vw = jnp.take(user_table, users, axis=0) * W.reshape(1, D)
    bias16 = jnp.broadcast_to(b.reshape(()), (16,)).astype(jnp.float32)

    run = pl.kernel(
        _body,
        out_type=jax.ShapeDtypeStruct((B, L), jnp.float32),
        mesh=plsc.VectorSubcoreMesh(
            core_axis_name="c", subcore_axis_name="s",
            num_cores=NC, num_subcores=NS),
        compiler_params=pltpu.CompilerParams(
            use_tc_tiling_on_sc=False, needs_layout_passes=False),
        scratch_types=[
            pltpu.VMEM((B // NW, D), jnp.float32),
            pltpu.VMEM((16,), jnp.float32),
            pltpu.VMEM((B // NW * L,), jnp.int32),
            pltpu.VMEM((NBUF, L, D), jnp.float32),
            pltpu.VMEM((B // NW, L), jnp.float32),
        ] + [pltpu.SemaphoreType.DMA] * NBUF,
    )
    return run(items_f, item_table, vw, bias16)
